# baseline (speedup 1.0000x reference)
import functools

import jax
import jax.numpy as jnp
from jax import lax
from jax.experimental import pallas as pl
from jax.experimental.pallas import tpu as pltpu
from jax.experimental.pallas import tpu_sc as plsc

N_NODES = 100000
HIDDEN = 64
N_EDGES = 1600000

NC = 2
NS = 16
LANES = 16

EB = 128
BLK = 4
ROWS_PER_W = 396
N_BLOCKS = ROWS_PER_W // BLK
N_TICK_IT = (N_BLOCKS + 4) // 3
EDGES_PER_W = ROWS_PER_W * EB
E_PAD = EDGES_PER_W * NC * NS
E_ROWS = E_PAD // EB

BLK_D = 6
N_BLOCKS_D = ROWS_PER_W // BLK_D
HALF_D = N_BLOCKS_D // 2

N_ACC = 100352
STRIPE = N_ACC // NS
TRASH = N_NODES

_SC_MESH = plsc.VectorSubcoreMesh(core_axis_name="c", subcore_axis_name="s")
_SC_PARAMS = pltpu.CompilerParams(use_tc_tiling_on_sc=False)


def _edge_sweep(table, ed3d, zeros, acc, idx_v, rows_v, isems, gsems, ssems,
                ebase, mult, g):
    def body(K, carry):
        for i in range(3):
            t = 3 * K + i
            slS = (i + 1) % 3
            slG = (i + 2) % 3
            slI = i

            @pl.when(jnp.logical_and(t >= 2, t <= N_BLOCKS + 1))
            def _(slS=slS):
                pltpu.make_async_copy(
                    zeros.at[pl.ds(0, BLK * EB)], rows_v.at[slS],
                    gsems[slS]).wait()
                for j in range(BLK):
                    pltpu.async_copy(
                        rows_v.at[slS, pl.ds(j * EB, EB)],
                        acc.at[idx_v.at[slS, j, 1]], ssems[slS], add=True)

            @pl.when(jnp.logical_and(t >= 1, t <= N_BLOCKS))
            def _(slG=slG):
                pltpu.make_async_copy(
                    ed3d.at[pl.ds(0, BLK)], idx_v.at[slG],
                    isems[slG]).wait()
                if mult != 1:
                    for j in range(BLK):
                        for v in range(EB // LANES):
                            q = pl.ds(v * LANES, LANES)
                            idx_v[slG, j, 0, q] = (
                                idx_v[slG, j, 0, q] * mult + g)
                for j in range(BLK):
                    pltpu.async_copy(
                        table.at[idx_v.at[slG, j, 0]],
                        rows_v.at[slG, pl.ds(j * EB, EB)], gsems[slG])

            @pl.when(t <= N_BLOCKS - 1)
            def _(slI=slI, t=t):
                @pl.when(t >= 3)
                def _():
                    pltpu.make_async_copy(
                        zeros.at[pl.ds(0, BLK * EB)], rows_v.at[slI],
                        ssems[slI]).wait()
                r0 = ebase + t * BLK
                pltpu.async_copy(
                    ed3d.at[pl.ds(r0, BLK)], idx_v.at[slI], isems[slI])
        return carry

    lax.fori_loop(0, N_TICK_IT, body, 0)
    for sl in range(3):
        pltpu.make_async_copy(
            zeros.at[pl.ds(0, BLK * EB)], rows_v.at[sl], ssems[sl]).wait()


_SPMM_SCRATCH = [
    pltpu.VMEM((3, BLK, 2, EB), jnp.int32),
    pltpu.VMEM((3, BLK * EB, LANES), jnp.float32),
    pltpu.VMEM_SHARED((N_ACC, LANES), jnp.float32),
] + [pltpu.SemaphoreType.DMA] * 9


def _make_spmm_kernel(mult: int, g: int):
    @functools.partial(
        pl.kernel,
        out_type=jax.ShapeDtypeStruct((NC, N_ACC, LANES), jnp.float32),
        mesh=_SC_MESH,
        compiler_params=_SC_PARAMS,
        scratch_types=_SPMM_SCRATCH,
    )
    def spmm(table, ed3d, zeros, out, idx_v, rows_v, acc,
             is0, is1, is2, gs0, gs1, gs2, ss0, ss1, ss2):
        c = lax.axis_index("c")
        s = lax.axis_index("s")
        w = c * NS + s
        base = s * STRIPE
        pltpu.sync_copy(zeros.at[pl.ds(base, STRIPE)],
                        acc.at[pl.ds(base, STRIPE)])
        plsc.subcore_barrier()
        _edge_sweep(table, ed3d, zeros, acc, idx_v, rows_v,
                    (is0, is1, is2), (gs0, gs1, gs2), (ss0, ss1, ss2),
                    w * ROWS_PER_W, mult, g)
        plsc.subcore_barrier()
        pltpu.sync_copy(acc.at[pl.ds(base, STRIPE)],
                        out.at[c, pl.ds(base, STRIPE)])

    return spmm


@functools.partial(
    pl.kernel,
    out_type=jax.ShapeDtypeStruct((NC, N_ACC, LANES), jnp.float32),
    mesh=_SC_MESH,
    compiler_params=_SC_PARAMS,
    scratch_types=[
        pltpu.VMEM((2, BLK_D, 2, EB), jnp.int32),
        pltpu.VMEM((EB, LANES), jnp.float32),
        pltpu.VMEM_SHARED((N_ACC, LANES), jnp.float32),
        pltpu.SemaphoreType.DMA,
        pltpu.SemaphoreType.DMA,
        pltpu.SemaphoreType.DMA,
        pltpu.SemaphoreType.DMA,
    ],
)
def _deg_kernel(ed3d, zeros, out, idx_v, ones_v, acc,
                isem0, isem1, ssem0, ssem1):
    c = lax.axis_index("c")
    s = lax.axis_index("s")
    w = c * NS + s
    isems = (isem0, isem1)
    ssems = (ssem0, ssem1)

    def ofill(i, carry):
        ones_v[i, :] = jnp.full((LANES,), 1.0, jnp.float32)
        return carry

    lax.fori_loop(0, EB, ofill, 0)
    base = s * STRIPE
    pltpu.sync_copy(zeros.at[pl.ds(base, STRIPE)], acc.at[pl.ds(base, STRIPE)])
    plsc.subcore_barrier()

    ebase = w * ROWS_PER_W

    def body(k, carry):
        ih = []
        for sl in range(2):
            @pl.when(k > 0)
            def _(sl=sl):
                for _j in range(BLK_D):
                    pltpu.make_async_copy(
                        zeros.at[pl.ds(0, EB)], ones_v, ssems[sl]).wait()
            r0 = ebase + (2 * k + sl) * BLK_D
            ih.append(pltpu.async_copy(
                ed3d.at[pl.ds(r0, BLK_D)], idx_v.at[sl], isems[sl]))
        for sl in range(2):
            ih[sl].wait()
            for j in range(BLK_D):
                pltpu.async_copy(
                    ones_v, acc.at[idx_v.at[sl, j, 1]], ssems[sl], add=True)
        return carry

    lax.fori_loop(0, HALF_D, body, 0)
    for sl in range(2):
        for _j in range(BLK_D):
            pltpu.make_async_copy(
                zeros.at[pl.ds(0, EB)], ones_v, ssems[sl]).wait()

    plsc.subcore_barrier()
    pltpu.sync_copy(acc.at[pl.ds(base, STRIPE)], out.at[c, pl.ds(base, STRIPE)])


BN = 2048
N_TC_BLOCKS = N_ACC // BN

_full = lambda shape: pl.BlockSpec(shape, lambda i: tuple(0 for _ in shape))
_rows = lambda wid: pl.BlockSpec((BN, wid), lambda i: (i, 0))
_part = pl.BlockSpec((NC, BN, LANES), lambda i: (0, i, 0))


def _stage0_body(degp, xv, dinv, xt):
    deg = degp[0, :, 0:1] + degp[1, :, 0:1] + 1.0
    di = lax.rsqrt(deg)
    dinv[...] = di
    xs = xv[...] * di
    xt[...] = jnp.concatenate(
        [xs, jnp.zeros((xs.shape[0], LANES - 2), jnp.float32)], axis=1)


def _stage0(degp, x_pad):
    return pl.pallas_call(
        _stage0_body,
        grid=(N_TC_BLOCKS,),
        in_specs=[_part, _rows(2)],
        out_specs=[_rows(1), _rows(LANES)],
        out_shape=[
            jax.ShapeDtypeStruct((N_ACC, 1), jnp.float32),
            jax.ShapeDtypeStruct((N_ACC, LANES), jnp.float32),
        ],
    )(degp, x_pad)


def _stage1_body(p, xt, dinv, W1, b1, out):
    di = dinv[...]
    sh = di * (p[0, :, 0:2] + p[1, :, 0:2] + xt[:, 0:2])
    z = jax.nn.relu(
        jnp.dot(sh, W1[...], preferred_element_type=jnp.float32) + b1[...])
    out[...] = di * z


def _stage1(p, xt, dinv, W1, b1):
    return pl.pallas_call(
        _stage1_body,
        grid=(N_TC_BLOCKS,),
        in_specs=[_part, _rows(LANES), _rows(1), _full((2, HIDDEN)),
                  _full((1, HIDDEN))],
        out_specs=_rows(HIDDEN),
        out_shape=jax.ShapeDtypeStruct((N_ACC, HIDDEN), jnp.float32),
    )(p, xt, dinv, W1, b1)


def _stage2_body(p0, p1, p2, p3, hprev, dinv, W, b, out):
    di = dinv[...]
    agg = jnp.concatenate(
        [p[0] + p[1] for p in (p0[...], p1[...], p2[...], p3[...])], axis=1)
    sh = di * (agg + hprev[...])
    z = jax.nn.relu(
        jnp.dot(sh, W[...], preferred_element_type=jnp.float32) + b[...])
    out[...] = di * z


def _stage2(ps, hprev, dinv, W, b):
    return pl.pallas_call(
        _stage2_body,
        grid=(N_TC_BLOCKS,),
        in_specs=[_part, _part, _part, _part, _rows(HIDDEN), _rows(1),
                  _full((HIDDEN, HIDDEN)), _full((1, HIDDEN))],
        out_specs=_rows(HIDDEN),
        out_shape=jax.ShapeDtypeStruct((N_ACC, HIDDEN), jnp.float32),
    )(*ps, hprev, dinv, W, b)


def _stage3_body(p0, p1, p2, p3, hprev, dinv, W3, b3, Wm1, bm1, Wm2, bm2,
                 out):
    di = dinv[...]
    agg = jnp.concatenate(
        [p[0] + p[1] for p in (p0[...], p1[...], p2[...], p3[...])], axis=1)
    sh = di * (agg + hprev[...])
    h3 = jnp.dot(sh, W3[...], preferred_element_type=jnp.float32) + b3[...]
    m = jax.nn.relu(
        jnp.dot(h3, Wm1[...], preferred_element_type=jnp.float32) + bm1[...])
    out[...] = jnp.dot(m, Wm2[...], preferred_element_type=jnp.float32) + bm2[...]


def _stage3(ps, hprev, dinv, W3, b3, Wm1, bm1, Wm2, bm2):
    return pl.pallas_call(
        _stage3_body,
        grid=(N_TC_BLOCKS,),
        in_specs=[_part, _part, _part, _part, _rows(HIDDEN), _rows(1),
                  _full((HIDDEN, HIDDEN)), _full((1, HIDDEN)),
                  _full((HIDDEN, HIDDEN)), _full((1, HIDDEN)),
                  _full((HIDDEN, 1)), _full((1, 1))],
        out_specs=_rows(1),
        out_shape=jax.ShapeDtypeStruct((N_ACC, 1), jnp.float32),
    )(*ps, hprev, dinv, W3, b3, Wm1, bm1, Wm2, bm2)


_SPMM_X = _make_spmm_kernel(1, 0)
_SPMM_G = [_make_spmm_kernel(4, g) for g in range(4)]


def kernel(x, edge_index, batch, W1, b1, W2, b2, W3, b3, Wm1, bm1, Wm2, bm2):
    del batch
    src = edge_index[0].astype(jnp.int32)
    dst = edge_index[1].astype(jnp.int32)
    pad = E_PAD - N_EDGES
    src2d = jnp.concatenate(
        [src, jnp.zeros((pad,), jnp.int32)]).reshape(E_ROWS, 1, EB)
    dst2d = jnp.concatenate(
        [dst, jnp.full((pad,), TRASH, jnp.int32)]).reshape(E_ROWS, 1, EB)
    ed3d = jnp.concatenate([src2d, dst2d], axis=1)
    x_pad = jnp.concatenate(
        [x, jnp.zeros((N_ACC - N_NODES, 2), jnp.float32)], axis=0)
    zeros = jnp.zeros((N_ACC, LANES), jnp.float32)

    degp = _deg_kernel(ed3d, zeros)
    dinv, xt = _stage0(degp, x_pad)

    agg1 = _SPMM_X(xt, ed3d, zeros)
    h1 = _stage1(agg1, xt, dinv, W1, b1.reshape(1, HIDDEN))

    t1 = h1.reshape(N_ACC * 4, LANES)
    p2 = [_SPMM_G[g](t1, ed3d, zeros) for g in range(4)]
    h2 = _stage2(p2, h1, dinv, W2, b2.reshape(1, HIDDEN))

    t2 = h2.reshape(N_ACC * 4, LANES)
    p3 = [_SPMM_G[g](t2, ed3d, zeros) for g in range(4)]
    out = _stage3(p3, h2, dinv, W3, b3.reshape(1, HIDDEN),
                  Wm1, bm1.reshape(1, HIDDEN), Wm2, bm2.reshape(1, 1))
    return out[:N_NODES]


# speedup vs baseline: 1.0415x; 1.0415x over previous
import functools

import jax
import jax.numpy as jnp
from jax import lax
from jax.experimental import pallas as pl
from jax.experimental.pallas import tpu as pltpu
from jax.experimental.pallas import tpu_sc as plsc

N_NODES = 100000
HIDDEN = 64
N_EDGES = 1600000

NC = 2
NS = 16
LANES = 16

EB = 128
BLK = 6
ROWS_PER_W = 396
N_BLOCKS = ROWS_PER_W // BLK
HALF = N_BLOCKS // 2
EDGES_PER_W = ROWS_PER_W * EB
E_PAD = EDGES_PER_W * NC * NS
E_ROWS = E_PAD // EB

BLK_D = 6
N_BLOCKS_D = ROWS_PER_W // BLK_D
HALF_D = N_BLOCKS_D // 2

N_ACC = 100352
STRIPE = N_ACC // NS
TRASH = N_NODES

_SC_MESH = plsc.VectorSubcoreMesh(core_axis_name="c", subcore_axis_name="s")
_SC_PARAMS = pltpu.CompilerParams(use_tc_tiling_on_sc=False)


def _edge_sweep(table, ed3d, zeros, acc, idx_v, rows_v, isems, gsems, ssems,
                ebase, mult, g):
    def body(k, carry):
        ih = []
        for sl in range(2):
            @pl.when(k > 0)
            def _(sl=sl):
                pltpu.make_async_copy(
                    zeros.at[pl.ds(0, BLK * EB)], rows_v.at[sl],
                    ssems[sl]).wait()
            r0 = ebase + (2 * k + sl) * BLK
            ih.append(pltpu.async_copy(
                ed3d.at[pl.ds(r0, BLK)], idx_v.at[sl], isems[sl]))
        gh = []
        for sl in range(2):
            ih[sl].wait()
            if mult != 1:
                for j in range(BLK):
                    for v in range(EB // LANES):
                        q = pl.ds(v * LANES, LANES)
                        idx_v[sl, j, 0, q] = idx_v[sl, j, 0, q] * mult + g
            gh.append([
                pltpu.async_copy(
                    table.at[idx_v.at[sl, j, 0]],
                    rows_v.at[sl, pl.ds(j * EB, EB)], gsems[sl])
                for j in range(BLK)
            ])
        for sl in range(2):
            for h in gh[sl]:
                h.wait()
            for j in range(BLK):
                pltpu.async_copy(
                    rows_v.at[sl, pl.ds(j * EB, EB)],
                    acc.at[idx_v.at[sl, j, 1]], ssems[sl], add=True)
        return carry

    lax.fori_loop(0, HALF, body, 0)
    for sl in range(2):
        pltpu.make_async_copy(
            zeros.at[pl.ds(0, BLK * EB)], rows_v.at[sl], ssems[sl]).wait()


_SPMM_SCRATCH = [
    pltpu.VMEM((2, BLK, 2, EB), jnp.int32),
    pltpu.VMEM((2, BLK * EB, LANES), jnp.float32),
    pltpu.VMEM_SHARED((N_ACC, LANES), jnp.float32),
] + [pltpu.SemaphoreType.DMA] * 6


def _make_spmm_kernel(mult: int, g: int):
    @functools.partial(
        pl.kernel,
        out_type=jax.ShapeDtypeStruct((NC, N_ACC, LANES), jnp.float32),
        mesh=_SC_MESH,
        compiler_params=_SC_PARAMS,
        scratch_types=_SPMM_SCRATCH,
    )
    def spmm(table, ed3d, zeros, out, idx_v, rows_v, acc,
             is0, is1, gs0, gs1, ss0, ss1):
        c = lax.axis_index("c")
        s = lax.axis_index("s")
        w = c * NS + s
        base = s * STRIPE
        pltpu.sync_copy(zeros.at[pl.ds(base, STRIPE)],
                        acc.at[pl.ds(base, STRIPE)])
        plsc.subcore_barrier()
        _edge_sweep(table, ed3d, zeros, acc, idx_v, rows_v,
                    (is0, is1), (gs0, gs1), (ss0, ss1),
                    w * ROWS_PER_W, mult, g)
        plsc.subcore_barrier()
        pltpu.sync_copy(acc.at[pl.ds(base, STRIPE)],
                        out.at[c, pl.ds(base, STRIPE)])

    return spmm


@functools.partial(
    pl.kernel,
    out_type=jax.ShapeDtypeStruct((NC, N_ACC, LANES), jnp.float32),
    mesh=_SC_MESH,
    compiler_params=_SC_PARAMS,
    scratch_types=[
        pltpu.VMEM((2, BLK_D, 2, EB), jnp.int32),
        pltpu.VMEM((EB, LANES), jnp.float32),
        pltpu.VMEM_SHARED((N_ACC, LANES), jnp.float32),
        pltpu.SemaphoreType.DMA,
        pltpu.SemaphoreType.DMA,
        pltpu.SemaphoreType.DMA,
        pltpu.SemaphoreType.DMA,
    ],
)
def _deg_kernel(ed3d, zeros, out, idx_v, ones_v, acc,
                isem0, isem1, ssem0, ssem1):
    c = lax.axis_index("c")
    s = lax.axis_index("s")
    w = c * NS + s
    isems = (isem0, isem1)
    ssems = (ssem0, ssem1)

    def ofill(i, carry):
        ones_v[i, :] = jnp.full((LANES,), 1.0, jnp.float32)
        return carry

    lax.fori_loop(0, EB, ofill, 0)
    base = s * STRIPE
    pltpu.sync_copy(zeros.at[pl.ds(base, STRIPE)], acc.at[pl.ds(base, STRIPE)])
    plsc.subcore_barrier()

    ebase = w * ROWS_PER_W

    def body(k, carry):
        ih = []
        for sl in range(2):
            @pl.when(k > 0)
            def _(sl=sl):
                for _j in range(BLK_D):
                    pltpu.make_async_copy(
                        zeros.at[pl.ds(0, EB)], ones_v, ssems[sl]).wait()
            r0 = ebase + (2 * k + sl) * BLK_D
            ih.append(pltpu.async_copy(
                ed3d.at[pl.ds(r0, BLK_D)], idx_v.at[sl], isems[sl]))
        for sl in range(2):
            ih[sl].wait()
            for j in range(BLK_D):
                pltpu.async_copy(
                    ones_v, acc.at[idx_v.at[sl, j, 1]], ssems[sl], add=True)
        return carry

    lax.fori_loop(0, HALF_D, body, 0)
    for sl in range(2):
        for _j in range(BLK_D):
            pltpu.make_async_copy(
                zeros.at[pl.ds(0, EB)], ones_v, ssems[sl]).wait()

    plsc.subcore_barrier()
    pltpu.sync_copy(acc.at[pl.ds(base, STRIPE)], out.at[c, pl.ds(base, STRIPE)])


BN = 2048
N_TC_BLOCKS = N_ACC // BN

_full = lambda shape: pl.BlockSpec(shape, lambda i: tuple(0 for _ in shape))
_rows = lambda wid: pl.BlockSpec((BN, wid), lambda i: (i, 0))
_part = pl.BlockSpec((NC, BN, LANES), lambda i: (0, i, 0))


def _stage0_body(degp, xv, dinv, xt):
    deg = degp[0, :, 0:1] + degp[1, :, 0:1] + 1.0
    di = lax.rsqrt(deg)
    dinv[...] = di
    xs = xv[...] * di
    xt[...] = jnp.concatenate(
        [xs, jnp.zeros((xs.shape[0], LANES - 2), jnp.float32)], axis=1)


def _stage0(degp, x_pad):
    return pl.pallas_call(
        _stage0_body,
        grid=(N_TC_BLOCKS,),
        in_specs=[_part, _rows(2)],
        out_specs=[_rows(1), _rows(LANES)],
        out_shape=[
            jax.ShapeDtypeStruct((N_ACC, 1), jnp.float32),
            jax.ShapeDtypeStruct((N_ACC, LANES), jnp.float32),
        ],
    )(degp, x_pad)


def _stage1_body(p, xt, dinv, W1, b1, out):
    di = dinv[...]
    sh = di * (p[0, :, 0:2] + p[1, :, 0:2] + xt[:, 0:2])
    z = jax.nn.relu(
        jnp.dot(sh, W1[...], preferred_element_type=jnp.float32) + b1[...])
    out[...] = di * z


def _stage1(p, xt, dinv, W1, b1):
    return pl.pallas_call(
        _stage1_body,
        grid=(N_TC_BLOCKS,),
        in_specs=[_part, _rows(LANES), _rows(1), _full((2, HIDDEN)),
                  _full((1, HIDDEN))],
        out_specs=_rows(HIDDEN),
        out_shape=jax.ShapeDtypeStruct((N_ACC, HIDDEN), jnp.float32),
    )(p, xt, dinv, W1, b1)


def _stage2_body(p0, p1, p2, p3, hprev, dinv, W, b, out):
    di = dinv[...]
    agg = jnp.concatenate(
        [p[0] + p[1] for p in (p0[...], p1[...], p2[...], p3[...])], axis=1)
    sh = di * (agg + hprev[...])
    z = jax.nn.relu(
        jnp.dot(sh, W[...], preferred_element_type=jnp.float32) + b[...])
    out[...] = di * z


def _stage2(ps, hprev, dinv, W, b):
    return pl.pallas_call(
        _stage2_body,
        grid=(N_TC_BLOCKS,),
        in_specs=[_part, _part, _part, _part, _rows(HIDDEN), _rows(1),
                  _full((HIDDEN, HIDDEN)), _full((1, HIDDEN))],
        out_specs=_rows(HIDDEN),
        out_shape=jax.ShapeDtypeStruct((N_ACC, HIDDEN), jnp.float32),
    )(*ps, hprev, dinv, W, b)


def _stage3_body(p0, p1, p2, p3, hprev, dinv, W3, b3, Wm1, bm1, Wm2, bm2,
                 out):
    di = dinv[...]
    agg = jnp.concatenate(
        [p[0] + p[1] for p in (p0[...], p1[...], p2[...], p3[...])], axis=1)
    sh = di * (agg + hprev[...])
    h3 = jnp.dot(sh, W3[...], preferred_element_type=jnp.float32) + b3[...]
    m = jax.nn.relu(
        jnp.dot(h3, Wm1[...], preferred_element_type=jnp.float32) + bm1[...])
    out[...] = jnp.dot(m, Wm2[...], preferred_element_type=jnp.float32) + bm2[...]


def _stage3(ps, hprev, dinv, W3, b3, Wm1, bm1, Wm2, bm2):
    return pl.pallas_call(
        _stage3_body,
        grid=(N_TC_BLOCKS,),
        in_specs=[_part, _part, _part, _part, _rows(HIDDEN), _rows(1),
                  _full((HIDDEN, HIDDEN)), _full((1, HIDDEN)),
                  _full((HIDDEN, HIDDEN)), _full((1, HIDDEN)),
                  _full((HIDDEN, 1)), _full((1, 1))],
        out_specs=_rows(1),
        out_shape=jax.ShapeDtypeStruct((N_ACC, 1), jnp.float32),
    )(*ps, hprev, dinv, W3, b3, Wm1, bm1, Wm2, bm2)


_SPMM_X = _make_spmm_kernel(1, 0)
_SPMM_G = [_make_spmm_kernel(4, g) for g in range(4)]


def kernel(x, edge_index, batch, W1, b1, W2, b2, W3, b3, Wm1, bm1, Wm2, bm2):
    del batch
    src = edge_index[0].astype(jnp.int32)
    dst = edge_index[1].astype(jnp.int32)
    pad = E_PAD - N_EDGES
    src2d = jnp.concatenate(
        [src, jnp.zeros((pad,), jnp.int32)]).reshape(E_ROWS, 1, EB)
    dst2d = jnp.concatenate(
        [dst, jnp.full((pad,), TRASH, jnp.int32)]).reshape(E_ROWS, 1, EB)
    ed3d = jnp.concatenate([src2d, dst2d], axis=1)
    x_pad = jnp.concatenate(
        [x, jnp.zeros((N_ACC - N_NODES, 2), jnp.float32)], axis=0)
    zeros = jnp.zeros((N_ACC, LANES), jnp.float32)

    degp = _deg_kernel(ed3d, zeros)
    dinv, xt = _stage0(degp, x_pad)

    agg1 = _SPMM_X(xt, ed3d, zeros)
    h1 = _stage1(agg1, xt, dinv, W1, b1.reshape(1, HIDDEN))

    t1 = h1.reshape(N_ACC * 4, LANES)
    p2 = [_SPMM_G[g](t1, ed3d, zeros) for g in range(4)]
    h2 = _stage2(p2, h1, dinv, W2, b2.reshape(1, HIDDEN))

    t2 = h2.reshape(N_ACC * 4, LANES)
    p3 = [_SPMM_G[g](t2, ed3d, zeros) for g in range(4)]
    out = _stage3(p3, h2, dinv, W3, b3.reshape(1, HIDDEN),
                  Wm1, bm1.reshape(1, HIDDEN), Wm2, bm2.reshape(1, 1))
    return out[:N_NODES]


# speedup vs baseline: 1.2477x; 1.1980x over previous
import functools

import jax
import jax.numpy as jnp
from jax import lax
from jax.experimental import pallas as pl
from jax.experimental.pallas import tpu as pltpu
from jax.experimental.pallas import tpu_sc as plsc

N_NODES = 100000
HIDDEN = 64
N_EDGES = 1600000

NC = 2
NS = 16
LANES = 16

EB = 128
BLK = 4
ROWS_PER_W = 392
N_BLOCKS = ROWS_PER_W // BLK
HALF = N_BLOCKS // 2
EDGES_PER_W = ROWS_PER_W * EB
E_PAD = EDGES_PER_W * NC * NS
E_ROWS = E_PAD // EB

BLK_D = 4
N_BLOCKS_D = ROWS_PER_W // BLK_D
HALF_D = N_BLOCKS_D // 2

N_ACC = 100352
STRIPE = N_ACC // NS
TRASH = N_NODES

_SC_MESH = plsc.VectorSubcoreMesh(core_axis_name="c", subcore_axis_name="s")
_SC_PARAMS = pltpu.CompilerParams(use_tc_tiling_on_sc=False)


def _edge_sweep(table, ed3d, zeros, acc, idx_v, rows_v, isems, gsems, ssems,
                ebase, mult, g):
    def body(k, carry):
        ih = []
        for sl in range(2):
            @pl.when(k > 0)
            def _(sl=sl):
                pltpu.make_async_copy(
                    zeros.at[pl.ds(0, BLK * EB)], rows_v.at[sl],
                    ssems[sl]).wait()
            r0 = ebase + (2 * k + sl) * BLK
            ih.append(pltpu.async_copy(
                ed3d.at[pl.ds(r0, BLK)], idx_v.at[sl], isems[sl]))
        gh = []
        for sl in range(2):
            ih[sl].wait()
            if mult != 1:
                for j in range(BLK):
                    for v in range(EB // LANES):
                        q = pl.ds(v * LANES, LANES)
                        idx_v[sl, j, 0, q] = idx_v[sl, j, 0, q] * mult + g
            gh.append([
                pltpu.async_copy(
                    table.at[idx_v.at[sl, j, 0]],
                    rows_v.at[sl, pl.ds(j * EB, EB)], gsems[sl])
                for j in range(BLK)
            ])
        for sl in range(2):
            for h in gh[sl]:
                h.wait()
            for j in range(BLK):
                pltpu.async_copy(
                    rows_v.at[sl, pl.ds(j * EB, EB)],
                    acc.at[idx_v.at[sl, j, 1]], ssems[sl], add=True)
        return carry

    lax.fori_loop(0, HALF, body, 0)
    for sl in range(2):
        pltpu.make_async_copy(
            zeros.at[pl.ds(0, BLK * EB)], rows_v.at[sl], ssems[sl]).wait()


_SPMM_SCRATCH = [
    pltpu.VMEM((2, BLK, 2, EB), jnp.int32),
    pltpu.VMEM((2, BLK * EB, LANES), jnp.float32),
    pltpu.VMEM_SHARED((N_ACC, LANES), jnp.float32),
] + [pltpu.SemaphoreType.DMA] * 6


def _make_spmm_kernel(mult: int, g: int):
    @functools.partial(
        pl.kernel,
        out_type=jax.ShapeDtypeStruct((NC, N_ACC, LANES), jnp.float32),
        mesh=_SC_MESH,
        compiler_params=_SC_PARAMS,
        scratch_types=_SPMM_SCRATCH,
    )
    def spmm(table, ed3d, zeros, out, idx_v, rows_v, acc,
             is0, is1, gs0, gs1, ss0, ss1):
        c = lax.axis_index("c")
        s = lax.axis_index("s")
        w = c * NS + s
        base = s * STRIPE
        pltpu.sync_copy(zeros.at[pl.ds(base, STRIPE)],
                        acc.at[pl.ds(base, STRIPE)])
        plsc.subcore_barrier()
        _edge_sweep(table, ed3d, zeros, acc, idx_v, rows_v,
                    (is0, is1), (gs0, gs1), (ss0, ss1),
                    w * ROWS_PER_W, mult, g)
        plsc.subcore_barrier()
        pltpu.sync_copy(acc.at[pl.ds(base, STRIPE)],
                        out.at[c, pl.ds(base, STRIPE)])

    return spmm


@functools.partial(
    pl.kernel,
    out_type=jax.ShapeDtypeStruct((NC, N_ACC, LANES), jnp.float32),
    mesh=_SC_MESH,
    compiler_params=_SC_PARAMS,
    scratch_types=[
        pltpu.VMEM((2, BLK_D, 2, EB), jnp.int32),
        pltpu.VMEM((EB, LANES), jnp.float32),
        pltpu.VMEM_SHARED((N_ACC, LANES), jnp.float32),
        pltpu.SemaphoreType.DMA,
        pltpu.SemaphoreType.DMA,
        pltpu.SemaphoreType.DMA,
        pltpu.SemaphoreType.DMA,
    ],
)
def _deg_kernel(ed3d, zeros, out, idx_v, ones_v, acc,
                isem0, isem1, ssem0, ssem1):
    c = lax.axis_index("c")
    s = lax.axis_index("s")
    w = c * NS + s
    isems = (isem0, isem1)
    ssems = (ssem0, ssem1)

    def ofill(i, carry):
        ones_v[i, :] = jnp.full((LANES,), 1.0, jnp.float32)
        return carry

    lax.fori_loop(0, EB, ofill, 0)
    base = s * STRIPE
    pltpu.sync_copy(zeros.at[pl.ds(base, STRIPE)], acc.at[pl.ds(base, STRIPE)])
    plsc.subcore_barrier()

    ebase = w * ROWS_PER_W

    def body(k, carry):
        ih = []
        for sl in range(2):
            @pl.when(k > 0)
            def _(sl=sl):
                for _j in range(BLK_D):
                    pltpu.make_async_copy(
                        zeros.at[pl.ds(0, EB)], ones_v, ssems[sl]).wait()
            r0 = ebase + (2 * k + sl) * BLK_D
            ih.append(pltpu.async_copy(
                ed3d.at[pl.ds(r0, BLK_D)], idx_v.at[sl], isems[sl]))
        for sl in range(2):
            ih[sl].wait()
            for j in range(BLK_D):
                pltpu.async_copy(
                    ones_v, acc.at[idx_v.at[sl, j, 1]], ssems[sl], add=True)
        return carry

    lax.fori_loop(0, HALF_D, body, 0)
    for sl in range(2):
        for _j in range(BLK_D):
            pltpu.make_async_copy(
                zeros.at[pl.ds(0, EB)], ones_v, ssems[sl]).wait()

    plsc.subcore_barrier()
    pltpu.sync_copy(acc.at[pl.ds(base, STRIPE)], out.at[c, pl.ds(base, STRIPE)])


BN = 2048
N_TC_BLOCKS = N_ACC // BN

_full = lambda shape: pl.BlockSpec(shape, lambda i: tuple(0 for _ in shape))
_rows = lambda wid: pl.BlockSpec((BN, wid), lambda i: (i, 0))
_part = pl.BlockSpec((NC, BN, LANES), lambda i: (0, i, 0))


def _stage0_body(degp, xv, dinv, xt):
    deg = degp[0, :, 0:1] + degp[1, :, 0:1] + 1.0
    di = lax.rsqrt(deg)
    dinv[...] = di
    xs = xv[...] * di
    xt[...] = jnp.concatenate(
        [xs, jnp.zeros((xs.shape[0], LANES - 2), jnp.float32)], axis=1)


def _stage0(degp, x_pad):
    return pl.pallas_call(
        _stage0_body,
        grid=(N_TC_BLOCKS,),
        in_specs=[_part, _rows(2)],
        out_specs=[_rows(1), _rows(LANES)],
        out_shape=[
            jax.ShapeDtypeStruct((N_ACC, 1), jnp.float32),
            jax.ShapeDtypeStruct((N_ACC, LANES), jnp.float32),
        ],
    )(degp, x_pad)


def _stage1_body(p, xt, dinv, W1, b1, out):
    di = dinv[...]
    sh = di * (p[0, :, 0:2] + p[1, :, 0:2] + xt[:, 0:2])
    z = jax.nn.relu(
        jnp.dot(sh, W1[...], preferred_element_type=jnp.float32) + b1[...])
    out[...] = di * z


def _stage1(p, xt, dinv, W1, b1):
    return pl.pallas_call(
        _stage1_body,
        grid=(N_TC_BLOCKS,),
        in_specs=[_part, _rows(LANES), _rows(1), _full((2, HIDDEN)),
                  _full((1, HIDDEN))],
        out_specs=_rows(HIDDEN),
        out_shape=jax.ShapeDtypeStruct((N_ACC, HIDDEN), jnp.float32),
    )(p, xt, dinv, W1, b1)


def _stage2_body(p0, p1, p2, p3, hprev, dinv, W, b, out):
    di = dinv[...]
    agg = jnp.concatenate(
        [p[0] + p[1] for p in (p0[...], p1[...], p2[...], p3[...])], axis=1)
    sh = di * (agg + hprev[...])
    z = jax.nn.relu(
        jnp.dot(sh, W[...], preferred_element_type=jnp.float32) + b[...])
    out[...] = di * z


def _stage2(ps, hprev, dinv, W, b):
    return pl.pallas_call(
        _stage2_body,
        grid=(N_TC_BLOCKS,),
        in_specs=[_part, _part, _part, _part, _rows(HIDDEN), _rows(1),
                  _full((HIDDEN, HIDDEN)), _full((1, HIDDEN))],
        out_specs=_rows(HIDDEN),
        out_shape=jax.ShapeDtypeStruct((N_ACC, HIDDEN), jnp.float32),
    )(*ps, hprev, dinv, W, b)


def _stage3_body(p0, p1, p2, p3, hprev, dinv, W3, b3, Wm1, bm1, Wm2, bm2,
                 out):
    di = dinv[...]
    agg = jnp.concatenate(
        [p[0] + p[1] for p in (p0[...], p1[...], p2[...], p3[...])], axis=1)
    sh = di * (agg + hprev[...])
    h3 = jnp.dot(sh, W3[...], preferred_element_type=jnp.float32) + b3[...]
    m = jax.nn.relu(
        jnp.dot(h3, Wm1[...], preferred_element_type=jnp.float32) + bm1[...])
    out[...] = jnp.dot(m, Wm2[...], preferred_element_type=jnp.float32) + bm2[...]


def _stage3(ps, hprev, dinv, W3, b3, Wm1, bm1, Wm2, bm2):
    return pl.pallas_call(
        _stage3_body,
        grid=(N_TC_BLOCKS,),
        in_specs=[_part, _part, _part, _part, _rows(HIDDEN), _rows(1),
                  _full((HIDDEN, HIDDEN)), _full((1, HIDDEN)),
                  _full((HIDDEN, HIDDEN)), _full((1, HIDDEN)),
                  _full((HIDDEN, 1)), _full((1, 1))],
        out_specs=_rows(1),
        out_shape=jax.ShapeDtypeStruct((N_ACC, 1), jnp.float32),
    )(*ps, hprev, dinv, W3, b3, Wm1, bm1, Wm2, bm2)


_SPMM_X = _make_spmm_kernel(1, 0)
_SPMM_G = [_make_spmm_kernel(4, g) for g in range(4)]


def kernel(x, edge_index, batch, W1, b1, W2, b2, W3, b3, Wm1, bm1, Wm2, bm2):
    del batch
    src = edge_index[0].astype(jnp.int32)
    dst = edge_index[1].astype(jnp.int32)
    pad = E_PAD - N_EDGES
    src2d = jnp.concatenate(
        [src, jnp.zeros((pad,), jnp.int32)]).reshape(E_ROWS, 1, EB)
    dst2d = jnp.concatenate(
        [dst, jnp.full((pad,), TRASH, jnp.int32)]).reshape(E_ROWS, 1, EB)
    ed3d = jnp.concatenate([src2d, dst2d], axis=1)
    x_pad = jnp.concatenate(
        [x, jnp.zeros((N_ACC - N_NODES, 2), jnp.float32)], axis=0)
    zeros = jnp.zeros((N_ACC, LANES), jnp.float32)

    degp = _deg_kernel(ed3d, zeros)
    dinv, xt = _stage0(degp, x_pad)

    agg1 = _SPMM_X(xt, ed3d, zeros)
    h1 = _stage1(agg1, xt, dinv, W1, b1.reshape(1, HIDDEN))

    t1 = h1.reshape(N_ACC * 4, LANES)
    p2 = [_SPMM_G[g](t1, ed3d, zeros) for g in range(4)]
    h2 = _stage2(p2, h1, dinv, W2, b2.reshape(1, HIDDEN))

    t2 = h2.reshape(N_ACC * 4, LANES)
    p3 = [_SPMM_G[g](t2, ed3d, zeros) for g in range(4)]
    out = _stage3(p3, h2, dinv, W3, b3.reshape(1, HIDDEN),
                  Wm1, bm1.reshape(1, HIDDEN), Wm2, bm2.reshape(1, 1))
    return out[:N_NODES]
